# baseline (speedup 1.0000x reference)
import jax
import jax.numpy as jnp
import numpy as np
from jax import lax
from jax.experimental import pallas as pl
from jax.experimental.pallas import tpu as pltpu
from jax.experimental.pallas import tpu_sc as plsc

VOCAB = 1000000
D_MODEL = 128
BATCH = 1024
SEQ = 200
MAX_POS = 512
BASE = 10000.0

NUM_WORKERS = 32
ROWS_PER_WORKER = BATCH // NUM_WORKERS
HALF = D_MODEL // 2
SPLIT = 104
NBUF = 3


def _rope_half_cache():
    inv_freq = 1.0 / (BASE ** (np.arange(0, D_MODEL, 2, dtype=np.float32) / D_MODEL))
    t = np.arange(MAX_POS, dtype=np.float32)
    freqs = np.einsum('i,j->ij', t, inv_freq)[:SEQ]
    cs = np.concatenate([np.cos(freqs), np.sin(freqs)], axis=0)
    return jnp.asarray(cs, dtype=jnp.float32)


def _sc_body(table_hbm, ids_hbm, cs_hbm, out_hbm,
             idx_v, rows_v, cs_v, gsem, osem, isem):
    wid = lax.axis_index("s") * 2 + lax.axis_index("c")
    base = wid * ROWS_PER_WORKER

    pltpu.sync_copy(cs_hbm, cs_v)
    for i in range(NBUF):
        pltpu.sync_copy(ids_hbm.at[pl.ds((base + i) * SEQ, SEQ)],
                        idx_v.at[pl.ds(i * SEQ, SEQ)])

    def gather_parts(b):
        yield idx_v.at[pl.ds(b * SEQ, SPLIT)], rows_v.at[b, pl.ds(0, SPLIT)]
        yield (idx_v.at[pl.ds(b * SEQ + SPLIT, SEQ - SPLIT)],
               rows_v.at[b, pl.ds(SPLIT, SEQ - SPLIT)])

    def start_gather(b):
        for idx, dst in gather_parts(b):
            pltpu.async_copy(table_hbm.at[idx], dst, gsem.at[b])

    def wait_gather(b):
        for idx, dst in gather_parts(b):
            pltpu.make_async_copy(table_hbm.at[idx], dst, gsem.at[b]).wait()

    def wait_out(b):
        pltpu.make_async_copy(rows_v.at[b], out_hbm.at[0], osem.at[b]).wait()

    def wait_idx(b):
        pltpu.make_async_copy(
            ids_hbm.at[pl.ds(0, SEQ)], idx_v.at[pl.ds(b * SEQ, SEQ)],
            isem.at[b]).wait()

    start_gather(0)
    start_gather(1)

    def per_row(j, carry):
        b = j % NBUF
        wait_gather(b)

        @pl.when(j < ROWS_PER_WORKER - NBUF)
        def _():
            pltpu.async_copy(ids_hbm.at[pl.ds((base + j + NBUF) * SEQ, SEQ)],
                             idx_v.at[pl.ds(b * SEQ, SEQ)], isem.at[b])

        @plsc.parallel_loop(0, SEQ, unroll=4)
        def _(t):
            for g in range(HALF // 16):
                h1 = rows_v[b, t, pl.ds(g * 16, 16)]
                h2 = rows_v[b, t, pl.ds(HALF + g * 16, 16)]
                cv = cs_v[t, pl.ds(g * 16, 16)]
                sv = cs_v[SEQ + t, pl.ds(g * 16, 16)]
                rows_v[b, t, pl.ds(g * 16, 16)] = h1 * cv - h2 * sv
                rows_v[b, t, pl.ds(HALF + g * 16, 16)] = h2 * cv + h1 * sv

        pltpu.async_copy(rows_v.at[b], out_hbm.at[base + j], osem.at[b])

        @pl.when(j < ROWS_PER_WORKER - 2)
        def _():
            b2 = (j + 2) % NBUF

            @pl.when(j >= 1)
            def _():
                wait_out(b2)
                wait_idx(b2)

            start_gather(b2)

        return carry

    lax.fori_loop(0, ROWS_PER_WORKER, per_row, 0)
    for b in range(NBUF):
        wait_out(b)


def kernel(input_ids, embed_table):
    cs_h = _rope_half_cache()

    mesh = plsc.VectorSubcoreMesh(core_axis_name="c", subcore_axis_name="s")
    run = pl.kernel(
        _sc_body,
        out_type=jax.ShapeDtypeStruct((BATCH, SEQ, D_MODEL), jnp.float32),
        mesh=mesh,
        scratch_types=[
            pltpu.VMEM((NBUF * SEQ,), jnp.int32),
            pltpu.VMEM((NBUF, SEQ, D_MODEL), jnp.float32),
            pltpu.VMEM((2 * SEQ, HALF), jnp.float32),
            pltpu.SemaphoreType.DMA((NBUF,)),
            pltpu.SemaphoreType.DMA((NBUF,)),
            pltpu.SemaphoreType.DMA((NBUF,)),
        ],
    )
    return run(embed_table, input_ids.reshape(-1), cs_h)


# speedup vs baseline: 1.0278x; 1.0278x over previous
import jax
import jax.numpy as jnp
import numpy as np
from jax import lax
from jax.experimental import pallas as pl
from jax.experimental.pallas import tpu as pltpu
from jax.experimental.pallas import tpu_sc as plsc

VOCAB = 1000000
D_MODEL = 128
BATCH = 1024
SEQ = 200
MAX_POS = 512
BASE = 10000.0

NUM_WORKERS = 32
ROWS_PER_WORKER = BATCH // NUM_WORKERS
HALF = D_MODEL // 2
GATHER_CHUNK = SEQ // 2
NBUF = 3


def _rope_half_cache():
    inv_freq = 1.0 / (BASE ** (np.arange(0, D_MODEL, 2, dtype=np.float32) / D_MODEL))
    t = np.arange(MAX_POS, dtype=np.float32)
    freqs = np.einsum('i,j->ij', t, inv_freq)[:SEQ]
    cs = np.concatenate([np.cos(freqs), np.sin(freqs)], axis=0)
    return jnp.asarray(cs, dtype=jnp.float32)


def _sc_body(table_hbm, ids_hbm, cs_hbm, out_hbm,
             idx_v, rows_v, cs_v, gsem, osem, isem):
    wid = lax.axis_index("s") * 2 + lax.axis_index("c")
    base = wid * ROWS_PER_WORKER

    pltpu.sync_copy(cs_hbm, cs_v)
    pltpu.sync_copy(ids_hbm.at[pl.ds(base, NBUF)], idx_v)

    def start_gather(j, b):
        for k in range(SEQ // GATHER_CHUNK):
            pltpu.async_copy(
                table_hbm.at[idx_v.at[b, k]],
                rows_v.at[b, pl.ds(k * GATHER_CHUNK, GATHER_CHUNK)],
                gsem.at[b],
            )

    def wait_gather(j, b):
        for k in range(SEQ // GATHER_CHUNK):
            pltpu.make_async_copy(
                table_hbm.at[idx_v.at[b, k]],
                rows_v.at[b, pl.ds(k * GATHER_CHUNK, GATHER_CHUNK)],
                gsem.at[b],
            ).wait()

    def wait_out(b):
        pltpu.make_async_copy(rows_v.at[b], out_hbm.at[0], osem.at[b]).wait()

    def wait_idx(b):
        pltpu.make_async_copy(
            ids_hbm.at[0], idx_v.at[b], isem.at[b]).wait()

    start_gather(0, 0)
    start_gather(1, 1)

    def per_row(j, carry):
        b = j % NBUF
        wait_gather(j, b)

        @pl.when(j < ROWS_PER_WORKER - NBUF)
        def _():
            pltpu.async_copy(ids_hbm.at[base + j + NBUF], idx_v.at[b],
                             isem.at[b])

        @plsc.parallel_loop(0, SEQ, unroll=4)
        def _(t):
            for g in range(HALF // 16):
                h1 = rows_v[b, t, pl.ds(g * 16, 16)]
                h2 = rows_v[b, t, pl.ds(HALF + g * 16, 16)]
                cv = cs_v[t, pl.ds(g * 16, 16)]
                sv = cs_v[SEQ + t, pl.ds(g * 16, 16)]
                rows_v[b, t, pl.ds(g * 16, 16)] = h1 * cv - h2 * sv
                rows_v[b, t, pl.ds(HALF + g * 16, 16)] = h2 * cv + h1 * sv

        pltpu.async_copy(rows_v.at[b], out_hbm.at[base + j], osem.at[b])

        @pl.when(j < ROWS_PER_WORKER - 2)
        def _():
            b2 = (j + 2) % NBUF

            @pl.when(j >= 1)
            def _():
                wait_out(b2)
                wait_idx(b2)

            start_gather(j + 2, b2)

        return carry

    lax.fori_loop(0, ROWS_PER_WORKER, per_row, 0)
    for b in range(NBUF):
        wait_out(b)


def kernel(input_ids, embed_table):
    cs_h = _rope_half_cache()
    ids = input_ids.reshape(BATCH, SEQ // GATHER_CHUNK, GATHER_CHUNK)

    mesh = plsc.VectorSubcoreMesh(core_axis_name="c", subcore_axis_name="s")
    run = pl.kernel(
        _sc_body,
        out_type=jax.ShapeDtypeStruct((BATCH, SEQ, D_MODEL), jnp.float32),
        mesh=mesh,
        scratch_types=[
            pltpu.VMEM((NBUF, SEQ // GATHER_CHUNK, GATHER_CHUNK), jnp.int32),
            pltpu.VMEM((NBUF, SEQ, D_MODEL), jnp.float32),
            pltpu.VMEM((2 * SEQ, HALF), jnp.float32),
            pltpu.SemaphoreType.DMA((NBUF,)),
            pltpu.SemaphoreType.DMA((NBUF,)),
            pltpu.SemaphoreType.DMA((NBUF,)),
        ],
    )
    return run(embed_table, ids, cs_h)


# speedup vs baseline: 1.0330x; 1.0051x over previous
import jax
import jax.numpy as jnp
import numpy as np
from jax import lax
from jax.experimental import pallas as pl
from jax.experimental.pallas import tpu as pltpu
from jax.experimental.pallas import tpu_sc as plsc

VOCAB = 1000000
D_MODEL = 128
BATCH = 1024
SEQ = 200
MAX_POS = 512
BASE = 10000.0

NUM_WORKERS = 32
ROWS_PER_WORKER = BATCH // NUM_WORKERS
HALF = D_MODEL // 2
GATHER_CHUNK = SEQ // 2
NBUF = 3


def _rope_half_cache():
    inv_freq = 1.0 / (BASE ** (np.arange(0, D_MODEL, 2, dtype=np.float32) / D_MODEL))
    t = np.arange(MAX_POS, dtype=np.float32)
    freqs = np.einsum('i,j->ij', t, inv_freq)[:SEQ]
    cs = np.concatenate([np.cos(freqs), np.sin(freqs)], axis=0)
    return jnp.asarray(cs, dtype=jnp.float32)


def _sc_body(table_hbm, ids_hbm, cs_hbm, out_hbm,
             idx_v, rows_v, cs_v, gsem, osem, isem):
    wid = lax.axis_index("s") * 2 + lax.axis_index("c")
    base = wid * ROWS_PER_WORKER

    pltpu.sync_copy(ids_hbm.at[pl.ds(base, NBUF)], idx_v)

    def start_gather(j, b):
        for k in range(SEQ // GATHER_CHUNK):
            pltpu.async_copy(
                table_hbm.at[idx_v.at[b, k]],
                rows_v.at[b, pl.ds(k * GATHER_CHUNK, GATHER_CHUNK)],
                gsem.at[b],
            )

    def wait_gather(j, b):
        for k in range(SEQ // GATHER_CHUNK):
            pltpu.make_async_copy(
                table_hbm.at[idx_v.at[b, k]],
                rows_v.at[b, pl.ds(k * GATHER_CHUNK, GATHER_CHUNK)],
                gsem.at[b],
            ).wait()

    def wait_out(b):
        pltpu.make_async_copy(rows_v.at[b], out_hbm.at[0], osem.at[b]).wait()

    def wait_idx(b):
        pltpu.make_async_copy(
            ids_hbm.at[0], idx_v.at[b], isem.at[b]).wait()

    start_gather(0, 0)
    start_gather(1, 1)
    pltpu.sync_copy(cs_hbm, cs_v)

    def per_row(j, carry):
        b = j % NBUF
        wait_gather(j, b)

        @pl.when(j < ROWS_PER_WORKER - NBUF)
        def _():
            pltpu.async_copy(ids_hbm.at[base + j + NBUF], idx_v.at[b],
                             isem.at[b])

        @plsc.parallel_loop(0, SEQ, unroll=4)
        def _(t):
            for g in range(HALF // 16):
                h1 = rows_v[b, t, pl.ds(g * 16, 16)]
                h2 = rows_v[b, t, pl.ds(HALF + g * 16, 16)]
                cv = cs_v[t, pl.ds(g * 16, 16)]
                sv = cs_v[SEQ + t, pl.ds(g * 16, 16)]
                rows_v[b, t, pl.ds(g * 16, 16)] = h1 * cv - h2 * sv
                rows_v[b, t, pl.ds(HALF + g * 16, 16)] = h2 * cv + h1 * sv

        pltpu.async_copy(rows_v.at[b], out_hbm.at[base + j], osem.at[b])

        @pl.when(j < ROWS_PER_WORKER - 2)
        def _():
            b2 = (j + 2) % NBUF

            @pl.when(j >= 1)
            def _():
                wait_out(b2)
                wait_idx(b2)

            start_gather(j + 2, b2)

        return carry

    lax.fori_loop(0, ROWS_PER_WORKER, per_row, 0)
    for b in range(NBUF):
        wait_out(b)


def kernel(input_ids, embed_table):
    cs_h = _rope_half_cache()
    ids = input_ids.reshape(BATCH, SEQ // GATHER_CHUNK, GATHER_CHUNK)

    mesh = plsc.VectorSubcoreMesh(core_axis_name="c", subcore_axis_name="s")
    run = pl.kernel(
        _sc_body,
        out_type=jax.ShapeDtypeStruct((BATCH, SEQ, D_MODEL), jnp.float32),
        mesh=mesh,
        scratch_types=[
            pltpu.VMEM((NBUF, SEQ // GATHER_CHUNK, GATHER_CHUNK), jnp.int32),
            pltpu.VMEM((NBUF, SEQ, D_MODEL), jnp.float32),
            pltpu.VMEM((2 * SEQ, HALF), jnp.float32),
            pltpu.SemaphoreType.DMA((NBUF,)),
            pltpu.SemaphoreType.DMA((NBUF,)),
            pltpu.SemaphoreType.DMA((NBUF,)),
        ],
    )
    return run(embed_table, ids, cs_h)
